# baseline (speedup 1.0000x reference)
import jax
import jax.numpy as jnp
from jax.experimental import pallas as pl
from jax.experimental.pallas import tpu as pltpu

_NP = 5120
_BLK = 128
_NBLK = _NP // _BLK
_IOU_THRESHOLD = 0.5


def _nms_kernel(lo_ref, hi_ref, data_ref, data_t_ref, out_ref, d_ref):
    s = data_ref[4:5, :]
    v = data_ref[6:7, :]

    x1 = data_ref[0:1, :]
    y1 = data_ref[1:2, :]
    x2 = data_ref[2:3, :]
    y2 = data_ref[3:4, :]
    cmax = jnp.maximum(jnp.maximum(x1, x2), jnp.maximum(y1, y2))
    cmax = jnp.where(v > 0, cmax, -jnp.inf)
    m = jnp.max(cmax)

    d_ref[...] = jnp.zeros((_NP, _NP), jnp.int8)

    def build_tile(j, r):
        c = j * _BLK
        cx1 = data_ref[0:1, pl.ds(c, _BLK)]
        cy1 = data_ref[1:2, pl.ds(c, _BLK)]
        cx2 = data_ref[2:3, pl.ds(c, _BLK)]
        cy2 = data_ref[3:4, pl.ds(c, _BLK)]
        cs = data_ref[4:5, pl.ds(c, _BLK)]
        ccls = data_ref[5:6, pl.ds(c, _BLK)]
        cv = data_ref[6:7, pl.ds(c, _BLK)]
        cid = data_ref[7:8, pl.ds(c, _BLK)]
        coff = ccls * (m + 1.0)
        cx1 = cx1 + coff
        cy1 = cy1 + coff
        cx2 = cx2 + coff
        cy2 = cy2 + coff
        carea = (cx2 - cx1) * (cy2 - cy1)

        rows = data_t_ref[pl.ds(r * _BLK, _BLK), :]
        boff = rows[:, 5:6] * (m + 1.0)
        bx1 = rows[:, 0:1] + boff
        by1 = rows[:, 1:2] + boff
        bx2 = rows[:, 2:3] + boff
        by2 = rows[:, 3:4] + boff
        bs = rows[:, 4:5]
        bv = rows[:, 6:7]
        bid = rows[:, 7:8]
        barea = (bx2 - bx1) * (by2 - by1)

        ix1 = jnp.maximum(bx1, cx1)
        iy1 = jnp.maximum(by1, cy1)
        ix2 = jnp.minimum(bx2, cx2)
        iy2 = jnp.minimum(by2, cy2)
        w = jnp.maximum(ix2 - ix1, 0.0)
        h = jnp.maximum(iy2 - iy1, 0.0)
        inter = w * h
        union = barea + carea - inter
        iou_gt = inter > _IOU_THRESHOLD * jnp.maximum(union, 1e-9)

        dom = (bs > cs) | ((bs == cs) & (bid < cid))
        tile = iou_gt & dom & (bv > 0) & (cv > 0)
        d_ref[pl.ds(r * _BLK, _BLK), pl.ds(c, _BLK)] = tile.astype(jnp.int8)
        return r

    def build_row(r, carry):
        jax.lax.fori_loop(lo_ref[r], hi_ref[r], build_tile, r)
        return carry

    jax.lax.fori_loop(0, _NBLK, build_row, 0)

    keep0 = jnp.where(v > 0, 1.0, 0.0).astype(jnp.float32)

    def cond(carry):
        changed, _ = carry
        return changed

    def body(carry):
        _, keep = carry
        ki8 = keep.astype(jnp.int8)
        supp = jax.lax.dot_general(
            ki8, d_ref[...],
            dimension_numbers=(((1,), (0,)), ((), ())),
            preferred_element_type=jnp.int32,
        )
        nk = jnp.where((supp == 0) & (v > 0), 1.0, 0.0).astype(jnp.float32)
        return jnp.any(nk != keep), nk

    _, keep = jax.lax.while_loop(cond, body, (jnp.asarray(True), keep0))

    out_ref[...] = jnp.concatenate(
        [keep, keep * s, jnp.zeros((6, _NP), jnp.float32)], axis=0)


def kernel(boxes, scores, class_ids):
    n = boxes.shape[0]
    order = jnp.argsort(class_ids, stable=True)
    sb = boxes[order]
    ss = scores[order]
    scls = class_ids[order]

    data = jnp.stack(
        [sb[:, 0], sb[:, 1], sb[:, 2], sb[:, 3], ss,
         scls.astype(jnp.float32),
         jnp.ones((n,), jnp.float32),
         order.astype(jnp.float32)], axis=0)
    data = jnp.pad(data, ((0, 0), (0, _NP - n)))
    data_t = data.T

    nclass = 32
    starts = jnp.searchsorted(scls, jnp.arange(nclass), side="left")
    ends = jnp.searchsorted(scls, jnp.arange(nclass), side="right")
    rstart = jnp.minimum(jnp.arange(_NBLK) * _BLK, n - 1)
    rend = jnp.minimum(jnp.arange(_NBLK) * _BLK + _BLK - 1, n - 1)
    cfirst = scls[rstart]
    clast = scls[rend]
    lo = (starts[cfirst] // _BLK).astype(jnp.int32)
    hi = ((ends[clast] + _BLK - 1) // _BLK).astype(jnp.int32)
    all_pad = jnp.arange(_NBLK) * _BLK >= n
    lo = jnp.where(all_pad, 0, lo)
    hi = jnp.where(all_pad, 0, hi)

    out = pl.pallas_call(
        _nms_kernel,
        out_shape=jax.ShapeDtypeStruct((8, _NP), jnp.float32),
        in_specs=[
            pl.BlockSpec(memory_space=pltpu.SMEM),
            pl.BlockSpec(memory_space=pltpu.SMEM),
            pl.BlockSpec(memory_space=pltpu.VMEM),
            pl.BlockSpec(memory_space=pltpu.VMEM),
        ],
        out_specs=pl.BlockSpec(memory_space=pltpu.VMEM),
        scratch_shapes=[pltpu.VMEM((_NP, _NP), jnp.int8)],
    )(lo, hi, data, data_t)

    keep_sorted = out[0, :n]
    kept_scores_sorted = out[1, :n]
    keep_mask = jnp.zeros((n,), jnp.float32).at[order].set(keep_sorted)
    kept_scores = jnp.zeros((n,), jnp.float32).at[order].set(kept_scores_sorted)
    return (keep_mask, kept_scores)


# speedup vs baseline: 1.4511x; 1.4511x over previous
import jax
import jax.numpy as jnp
from jax.experimental import pallas as pl
from jax.experimental.pallas import tpu as pltpu

_NP = 5120
_BLK = 128
_NBLK = _NP // _BLK
_IOU_THRESHOLD = 0.5


def _nms_kernel(lo_ref, hi_ref, data_ref, data_t_ref, out_ref, d_ref):
    s = data_ref[4:5, :]
    v = data_ref[6:7, :]

    x1 = data_ref[0:1, :]
    y1 = data_ref[1:2, :]
    x2 = data_ref[2:3, :]
    y2 = data_ref[3:4, :]
    cmax = jnp.maximum(jnp.maximum(x1, x2), jnp.maximum(y1, y2))
    cmax = jnp.where(v > 0, cmax, -jnp.inf)
    m = jnp.max(cmax)

    d_ref[...] = jnp.zeros((_NP, _NP), jnp.int8)

    def build_tile(j, r):
        c = j * _BLK
        ccls = data_ref[5:6, pl.ds(c, _BLK)]
        coff = ccls * (m + 1.0)
        cx1 = data_ref[0:1, pl.ds(c, _BLK)] + coff
        cy1 = data_ref[1:2, pl.ds(c, _BLK)] + coff
        cx2 = data_ref[2:3, pl.ds(c, _BLK)] + coff
        cy2 = data_ref[3:4, pl.ds(c, _BLK)] + coff
        cs = data_ref[4:5, pl.ds(c, _BLK)]
        cid = data_ref[7:8, pl.ds(c, _BLK)]
        carea = (cx2 - cx1) * (cy2 - cy1)

        rows = data_t_ref[pl.ds(r * _BLK, _BLK), :]
        boff = rows[:, 5:6] * (m + 1.0)
        bx1 = rows[:, 0:1] + boff
        by1 = rows[:, 1:2] + boff
        bx2 = rows[:, 2:3] + boff
        by2 = rows[:, 3:4] + boff
        bs = rows[:, 4:5]
        bid = rows[:, 7:8]
        barea = (bx2 - bx1) * (by2 - by1)

        ix1 = jnp.maximum(bx1, cx1)
        iy1 = jnp.maximum(by1, cy1)
        ix2 = jnp.minimum(bx2, cx2)
        iy2 = jnp.minimum(by2, cy2)
        w = jnp.maximum(ix2 - ix1, 0.0)
        h = jnp.maximum(iy2 - iy1, 0.0)
        inter = w * h
        asum = barea + carea
        iou_gt = 3.0 * inter > asum

        dom = (bs > cs) | ((bs == cs) & (bid < cid))
        tile = iou_gt & dom
        d_ref[pl.ds(r * _BLK, _BLK), pl.ds(c, _BLK)] = tile.astype(jnp.int8)
        return r

    def build_row(r, carry):
        jax.lax.fori_loop(lo_ref[r], hi_ref[r], build_tile, r)
        return carry

    jax.lax.fori_loop(0, _NBLK, build_row, 0)

    keep0 = jnp.ones((1, _NP), jnp.float32)

    def cond(carry):
        changed, _ = carry
        return changed

    def body(carry):
        _, keep = carry
        ki8 = keep.astype(jnp.int8)
        supp = jax.lax.dot_general(
            ki8, d_ref[...],
            dimension_numbers=(((1,), (0,)), ((), ())),
            preferred_element_type=jnp.int32,
        )
        nk = jnp.where(supp == 0, 1.0, 0.0).astype(jnp.float32)
        return jnp.any(nk != keep), nk

    _, keep = jax.lax.while_loop(cond, body, (jnp.asarray(True), keep0))

    out_ref[...] = jnp.concatenate(
        [keep, jnp.zeros((7, _NP), jnp.float32)], axis=0)


def kernel(boxes, scores, class_ids):
    n = boxes.shape[0]
    order = jnp.argsort(class_ids, stable=True)
    base = jnp.concatenate(
        [boxes,
         scores[:, None],
         class_ids.astype(jnp.float32)[:, None],
         jnp.ones((n, 1), jnp.float32),
         jnp.arange(n, dtype=jnp.float32)[:, None]], axis=1)
    sorted8 = base[order]
    data_t = jnp.pad(sorted8, ((0, _NP - n), (0, 0)))
    data = data_t.T

    scls = sorted8[:, 5]
    nclass = 32
    cgrid = jnp.arange(nclass, dtype=jnp.float32)
    starts = jnp.searchsorted(scls, cgrid, side="left")
    ends = jnp.searchsorted(scls, cgrid, side="right")
    rstart = jnp.minimum(jnp.arange(_NBLK) * _BLK, n - 1)
    rend = jnp.minimum(jnp.arange(_NBLK) * _BLK + _BLK - 1, n - 1)
    cfirst = scls[rstart].astype(jnp.int32)
    clast = scls[rend].astype(jnp.int32)
    lo = (starts[cfirst] // _BLK).astype(jnp.int32)
    hi = ((ends[clast] + _BLK - 1) // _BLK).astype(jnp.int32)
    all_pad = jnp.arange(_NBLK) * _BLK >= n
    lo = jnp.where(all_pad, 0, lo)
    hi = jnp.where(all_pad, 0, hi)

    out = pl.pallas_call(
        _nms_kernel,
        out_shape=jax.ShapeDtypeStruct((8, _NP), jnp.float32),
        in_specs=[
            pl.BlockSpec(memory_space=pltpu.SMEM),
            pl.BlockSpec(memory_space=pltpu.SMEM),
            pl.BlockSpec(memory_space=pltpu.VMEM),
            pl.BlockSpec(memory_space=pltpu.VMEM),
        ],
        out_specs=pl.BlockSpec(memory_space=pltpu.VMEM),
        scratch_shapes=[pltpu.VMEM((_NP, _NP), jnp.int8)],
    )(lo, hi, data, data_t)

    keep_sorted = out[0, :n]
    keep_mask = jnp.zeros((n,), jnp.float32).at[order].set(keep_sorted)
    kept_scores = scores * keep_mask
    return (keep_mask, kept_scores)


# speedup vs baseline: 1.6373x; 1.1283x over previous
import jax
import jax.numpy as jnp
from jax.experimental import pallas as pl
from jax.experimental.pallas import tpu as pltpu

_NP = 5120
_BLK = 128
_NBLK = _NP // _BLK
_IOU_THRESHOLD = 0.5


def _nms_kernel(lo_ref, hi_ref, data_ref, data_t_ref, out_ref, d_ref):
    s = data_ref[4:5, :]
    v = data_ref[6:7, :]

    x1 = data_ref[0:1, :]
    y1 = data_ref[1:2, :]
    x2 = data_ref[2:3, :]
    y2 = data_ref[3:4, :]
    cmax = jnp.maximum(jnp.maximum(x1, x2), jnp.maximum(y1, y2))
    cmax = jnp.where(v > 0, cmax, -jnp.inf)
    m = jnp.max(cmax)

    d_ref[...] = jnp.zeros((_NP, _NP), jnp.int8)

    def _tile_parts(r, c):
        ccls = data_ref[5:6, pl.ds(c, _BLK)]
        coff = ccls * (m + 1.0)
        cx1 = data_ref[0:1, pl.ds(c, _BLK)] + coff
        cy1 = data_ref[1:2, pl.ds(c, _BLK)] + coff
        cx2 = data_ref[2:3, pl.ds(c, _BLK)] + coff
        cy2 = data_ref[3:4, pl.ds(c, _BLK)] + coff
        cs = data_ref[4:5, pl.ds(c, _BLK)]
        cid = data_ref[7:8, pl.ds(c, _BLK)]
        carea = (cx2 - cx1) * (cy2 - cy1)

        rows = data_t_ref[pl.ds(r * _BLK, _BLK), :]
        boff = rows[:, 5:6] * (m + 1.0)
        bx1 = rows[:, 0:1] + boff
        by1 = rows[:, 1:2] + boff
        bx2 = rows[:, 2:3] + boff
        by2 = rows[:, 3:4] + boff
        bs = rows[:, 4:5]
        bid = rows[:, 7:8]
        barea = (bx2 - bx1) * (by2 - by1)

        ix1 = jnp.maximum(bx1, cx1)
        iy1 = jnp.maximum(by1, cy1)
        ix2 = jnp.minimum(bx2, cx2)
        iy2 = jnp.minimum(by2, cy2)
        w = jnp.maximum(ix2 - ix1, 0.0)
        h = jnp.maximum(iy2 - iy1, 0.0)
        inter = w * h
        asum = barea + carea
        iou_gt = 3.0 * inter > asum

        dom = (bs > cs) | ((bs == cs) & (bid < cid))
        return iou_gt, dom

    def build_offdiag(j, r):
        c = j * _BLK
        iou_gt, dom = _tile_parts(r, c)
        up = (iou_gt & dom).astype(jnp.int8)
        down = (iou_gt & (~dom)).astype(jnp.int8)
        d_ref[pl.ds(r * _BLK, _BLK), pl.ds(c, _BLK)] = up
        d_ref[pl.ds(c, _BLK), pl.ds(r * _BLK, _BLK)] = down.T
        return r

    def build_row(r, carry):
        iou_gt, dom = _tile_parts(r, r * _BLK)
        d_ref[pl.ds(r * _BLK, _BLK), pl.ds(r * _BLK, _BLK)] = (
            (iou_gt & dom).astype(jnp.int8))
        jax.lax.fori_loop(jnp.maximum(lo_ref[r], r + 1), hi_ref[r],
                          build_offdiag, r)
        return carry

    jax.lax.fori_loop(0, _NBLK, build_row, 0)

    keep0 = jnp.ones((1, _NP), jnp.float32)

    def cond(carry):
        changed, _ = carry
        return changed

    def body(carry):
        _, keep = carry
        ki8 = keep.astype(jnp.int8)
        supp = jax.lax.dot_general(
            ki8, d_ref[...],
            dimension_numbers=(((1,), (0,)), ((), ())),
            preferred_element_type=jnp.int32,
        )
        nk = jnp.where(supp == 0, 1.0, 0.0).astype(jnp.float32)
        return jnp.any(nk != keep), nk

    _, keep = jax.lax.while_loop(cond, body, (jnp.asarray(True), keep0))

    out_ref[...] = jnp.concatenate(
        [keep, jnp.zeros((7, _NP), jnp.float32)], axis=0)


def kernel(boxes, scores, class_ids):
    n = boxes.shape[0]
    order = jnp.argsort(class_ids, stable=True)
    base = jnp.concatenate(
        [boxes,
         scores[:, None],
         class_ids.astype(jnp.float32)[:, None],
         jnp.ones((n, 1), jnp.float32),
         jnp.arange(n, dtype=jnp.float32)[:, None]], axis=1)
    sorted8 = base[order]
    data_t = jnp.pad(sorted8, ((0, _NP - n), (0, 0)))
    data = data_t.T

    scls = sorted8[:, 5]
    nclass = 32
    cgrid = jnp.arange(nclass, dtype=jnp.float32)
    starts = jnp.searchsorted(scls, cgrid, side="left")
    ends = jnp.searchsorted(scls, cgrid, side="right")
    rstart = jnp.minimum(jnp.arange(_NBLK) * _BLK, n - 1)
    rend = jnp.minimum(jnp.arange(_NBLK) * _BLK + _BLK - 1, n - 1)
    cfirst = scls[rstart].astype(jnp.int32)
    clast = scls[rend].astype(jnp.int32)
    lo = (starts[cfirst] // _BLK).astype(jnp.int32)
    hi = ((ends[clast] + _BLK - 1) // _BLK).astype(jnp.int32)
    all_pad = jnp.arange(_NBLK) * _BLK >= n
    lo = jnp.where(all_pad, 0, lo)
    hi = jnp.where(all_pad, 0, hi)

    out = pl.pallas_call(
        _nms_kernel,
        out_shape=jax.ShapeDtypeStruct((8, _NP), jnp.float32),
        in_specs=[
            pl.BlockSpec(memory_space=pltpu.SMEM),
            pl.BlockSpec(memory_space=pltpu.SMEM),
            pl.BlockSpec(memory_space=pltpu.VMEM),
            pl.BlockSpec(memory_space=pltpu.VMEM),
        ],
        out_specs=pl.BlockSpec(memory_space=pltpu.VMEM),
        scratch_shapes=[pltpu.VMEM((_NP, _NP), jnp.int8)],
    )(lo, hi, data, data_t)

    keep_sorted = out[0, :n]
    keep_mask = jnp.zeros((n,), jnp.float32).at[order].set(keep_sorted)
    kept_scores = scores * keep_mask
    return (keep_mask, kept_scores)


# speedup vs baseline: 1.8234x; 1.1137x over previous
import jax
import jax.numpy as jnp
from jax.experimental import pallas as pl
from jax.experimental.pallas import tpu as pltpu

_NP = 5120
_BLK = 128
_NBLK = _NP // _BLK
_IOU_THRESHOLD = 0.5


def _nms_kernel(lo_ref, hi_ref, data_ref, data_t_ref, out_ref, d_ref):
    s = data_ref[4:5, :]
    v = data_ref[6:7, :]

    x1 = data_ref[0:1, :]
    y1 = data_ref[1:2, :]
    x2 = data_ref[2:3, :]
    y2 = data_ref[3:4, :]
    cmax = jnp.maximum(jnp.maximum(x1, x2), jnp.maximum(y1, y2))
    cmax = jnp.where(v > 0, cmax, -jnp.inf)
    m = jnp.max(cmax)

    d_ref[...] = jnp.zeros((_NP, _NP), jnp.int8)

    def _tile_parts(r, c):
        ccls = data_ref[5:6, pl.ds(c, _BLK)]
        coff = ccls * (m + 1.0)
        cx1 = data_ref[0:1, pl.ds(c, _BLK)] + coff
        cy1 = data_ref[1:2, pl.ds(c, _BLK)] + coff
        cx2 = data_ref[2:3, pl.ds(c, _BLK)] + coff
        cy2 = data_ref[3:4, pl.ds(c, _BLK)] + coff
        cs = data_ref[4:5, pl.ds(c, _BLK)]
        cid = data_ref[7:8, pl.ds(c, _BLK)]
        carea = (cx2 - cx1) * (cy2 - cy1)

        rows = data_t_ref[pl.ds(r * _BLK, _BLK), :]
        boff = rows[:, 5:6] * (m + 1.0)
        bx1 = rows[:, 0:1] + boff
        by1 = rows[:, 1:2] + boff
        bx2 = rows[:, 2:3] + boff
        by2 = rows[:, 3:4] + boff
        bs = rows[:, 4:5]
        bid = rows[:, 7:8]
        barea = (bx2 - bx1) * (by2 - by1)

        ix1 = jnp.maximum(bx1, cx1)
        iy1 = jnp.maximum(by1, cy1)
        ix2 = jnp.minimum(bx2, cx2)
        iy2 = jnp.minimum(by2, cy2)
        w = jnp.maximum(ix2 - ix1, 0.0)
        h = jnp.maximum(iy2 - iy1, 0.0)
        inter = w * h
        asum = barea + carea
        iou_gt = 3.0 * inter > asum

        dom = (bs > cs) | ((bs == cs) & (bid < cid))
        return iou_gt, dom

    def build_offdiag(j, r):
        c = j * _BLK
        iou_gt, dom = _tile_parts(r, c)
        up = (iou_gt & dom).astype(jnp.int8)
        down = (iou_gt & (~dom)).astype(jnp.int8)
        d_ref[pl.ds(r * _BLK, _BLK), pl.ds(c, _BLK)] = up
        d_ref[pl.ds(c, _BLK), pl.ds(r * _BLK, _BLK)] = down.T
        return r

    def build_row(r, carry):
        iou_gt, dom = _tile_parts(r, r * _BLK)
        d_ref[pl.ds(r * _BLK, _BLK), pl.ds(r * _BLK, _BLK)] = (
            (iou_gt & dom).astype(jnp.int8))
        jax.lax.fori_loop(jnp.maximum(lo_ref[r], r + 1), hi_ref[r],
                          build_offdiag, r)
        return carry

    jax.lax.fori_loop(0, _NBLK, build_row, 0)

    keep0 = jnp.ones((1, _NP), jnp.float32)

    def sweep(keep):
        ki8 = keep.astype(jnp.int8)
        supp = jax.lax.dot_general(
            ki8, d_ref[...],
            dimension_numbers=(((1,), (0,)), ((), ())),
            preferred_element_type=jnp.int32,
        )
        return jnp.where(supp == 0, 1.0, 0.0).astype(jnp.float32)

    def cond(carry):
        changed, _ = carry
        return changed

    def body(carry):
        _, keep = carry
        nk = sweep(keep)
        return jnp.any(nk != keep), nk

    _, keep = jax.lax.while_loop(cond, body, (jnp.asarray(True), keep0))

    out_ref[...] = jnp.concatenate(
        [keep, jnp.zeros((7, _NP), jnp.float32)], axis=0)


def kernel(boxes, scores, class_ids):
    n = boxes.shape[0]
    order = jnp.argsort(class_ids, stable=True)
    base = jnp.concatenate(
        [boxes,
         scores[:, None],
         class_ids.astype(jnp.float32)[:, None],
         jnp.ones((n, 1), jnp.float32),
         jnp.arange(n, dtype=jnp.float32)[:, None]], axis=1)
    perm = (order[:, None] == jnp.arange(n)[None, :]).astype(jnp.bfloat16)
    sorted8 = jax.lax.dot_general(
        perm, base, (((1,), (0,)), ((), ())),
        preferred_element_type=jnp.float32)
    data_t = jnp.pad(sorted8, ((0, _NP - n), (0, 0)))
    data = data_t.T

    scls = sorted8[:, 5]
    nclass = 32
    cgrid = jnp.arange(nclass, dtype=jnp.float32)
    starts = jnp.searchsorted(scls, cgrid, side="left")
    ends = jnp.searchsorted(scls, cgrid, side="right")
    rstart = jnp.minimum(jnp.arange(_NBLK) * _BLK, n - 1)
    rend = jnp.minimum(jnp.arange(_NBLK) * _BLK + _BLK - 1, n - 1)
    cfirst = scls[rstart].astype(jnp.int32)
    clast = scls[rend].astype(jnp.int32)
    lo = (starts[cfirst] // _BLK).astype(jnp.int32)
    hi = ((ends[clast] + _BLK - 1) // _BLK).astype(jnp.int32)
    all_pad = jnp.arange(_NBLK) * _BLK >= n
    lo = jnp.where(all_pad, 0, lo)
    hi = jnp.where(all_pad, 0, hi)

    out = pl.pallas_call(
        _nms_kernel,
        out_shape=jax.ShapeDtypeStruct((8, _NP), jnp.float32),
        in_specs=[
            pl.BlockSpec(memory_space=pltpu.SMEM),
            pl.BlockSpec(memory_space=pltpu.SMEM),
            pl.BlockSpec(memory_space=pltpu.VMEM),
            pl.BlockSpec(memory_space=pltpu.VMEM),
        ],
        out_specs=pl.BlockSpec(memory_space=pltpu.VMEM),
        scratch_shapes=[pltpu.VMEM((_NP, _NP), jnp.int8)],
    )(lo, hi, data, data_t)

    keep_sorted = out[0, :n]
    keep_mask = jax.lax.dot_general(
        keep_sorted[None, :].astype(jnp.bfloat16), perm,
        (((1,), (0,)), ((), ())),
        preferred_element_type=jnp.float32)[0]
    kept_scores = scores * keep_mask
    return (keep_mask, kept_scores)
